# baseline (speedup 1.0000x reference)
import jax
import jax.numpy as jnp
from jax import lax
from jax.experimental import pallas as pl
from jax.experimental.pallas import tpu as pltpu
from jax.experimental.pallas import tpu_sc as plsc

N = 10000
E = 160000
D = 128
R = 8
NPAD = 10240
NT = 32
K = 128
NB = E // K

_f32 = jnp.float32
_i32 = jnp.int32


def _mesh():
    return plsc.VectorSubcoreMesh(core_axis_name="c", subcore_axis_name="s")


_SC_PARAMS = pltpu.CompilerParams(needs_layout_passes=False)


def _deg_kernel(src_hbm, dst_hbm, eid_hbm, zn_hbm, out_o, out_i, edata_out,
                acc_o, acc_i, srcv, dstv, eidv, packa, packb, sem):
    wid = lax.axis_index("s") * 2 + lax.axis_index("c")
    pltpu.sync_copy(zn_hbm, acc_o)
    pltpu.sync_copy(zn_hbm, acc_i)
    ones = jnp.full((16,), 1.0, _f32)

    def batch(b, carry):
        bid = b * NT + wid

        @pl.when(bid < NB)
        def _():
            base = bid * K
            pltpu.sync_copy(src_hbm.at[pl.ds(base, K)], srcv)
            pltpu.sync_copy(dst_hbm.at[pl.ds(base, K)], dstv)
            pltpu.sync_copy(eid_hbm.at[pl.ds(base, K)], eidv)

            for c in range(K // 16):
                sl = pl.ds(c * 16, 16)
                hsl = pl.ds((c % 4) * 16, 16)
                s16 = srcv[sl]
                d16 = dstv[sl]
                e16 = eidv[sl]
                plsc.addupdate_scatter(acc_o, [s16], ones)
                plsc.addupdate_scatter(acc_i, [d16], ones)
                pk = packa if c < 4 else packb
                pk[0, hsl] = s16 * R + e16
                pk[1, hsl] = d16 * R + e16
                pk[2, hsl] = d16
                pk[3, hsl] = s16

            pltpu.sync_copy(packa, edata_out.at[2 * bid])
            pltpu.sync_copy(packb, edata_out.at[2 * bid + 1])

        return carry

    lax.fori_loop(0, (NB + NT - 1) // NT, batch, 0)
    pltpu.sync_copy(acc_o, out_o.at[wid])
    pltpu.sync_copy(acc_i, out_i.at[wid])


def _deg_call(src, dst, eid, zn):
    fn = pl.kernel(
        _deg_kernel,
        out_type=[
            jax.ShapeDtypeStruct((NT, NPAD), _f32),
            jax.ShapeDtypeStruct((NT, NPAD), _f32),
            jax.ShapeDtypeStruct((NBA, 4, KA), _i32),
        ],
        mesh=_mesh(),
        compiler_params=_SC_PARAMS,
        scratch_types=[
            pltpu.VMEM((NPAD,), _f32),
            pltpu.VMEM((NPAD,), _f32),
            pltpu.VMEM((K,), _i32),
            pltpu.VMEM((K,), _i32),
            pltpu.VMEM((K,), _i32),
            pltpu.VMEM((4, KA), _i32),
            pltpu.VMEM((4, KA), _i32),
            pltpu.SemaphoreType.DMA,
        ],
    )
    return fn(src, dst, eid, zn)


def _score_kernel(edata_hbm, sh_hbm, sf_hbm, seb_hbm, m_hbm,
                  zn_hbm, ee_out, den_out,
                  accd, pack0, pack1, va0, va1, vb0, vb1, sebv0, sebv1,
                  eev0, eev1, mv, lsem0, lsem1, gsem0, gsem1):
    packs = (pack0, pack1)
    vas = (va0, va1)
    vbs = (vb0, vb1)
    sebvs = (sebv0, sebv1)
    eevs = (eev0, eev1)
    lsems = (lsem0, lsem1)
    gsems = (gsem0, gsem1)
    wid = lax.axis_index("s") * 2 + lax.axis_index("c")
    pltpu.sync_copy(zn_hbm, accd)
    pltpu.sync_copy(m_hbm, mv)
    m16 = mv[pl.ds(0, 16)]

    nbt = (NBA + NT - 1) // NT

    def issue_wave_a(i, par):
        bid = i * NT + wid

        @pl.when(bid < NBA)
        def _():
            pltpu.async_copy(edata_hbm.at[bid], packs[par], lsems[par])
            pltpu.async_copy(seb_hbm.at[pl.ds(bid * KA, KA)], sebvs[par],
                             lsems[par])

    def wait_wave_a(par):
        pltpu.make_async_copy(edata_hbm.at[0], packs[par], lsems[par]).wait()
        pltpu.make_async_copy(seb_hbm.at[pl.ds(0, KA)], sebvs[par],
                              lsems[par]).wait()

    def issue_wave_b(par):
        pltpu.async_copy(sh_hbm.at[packs[par].at[0]], vas[par], gsems[par])
        pltpu.async_copy(sf_hbm.at[packs[par].at[1]], vbs[par], gsems[par])

    def wait_wave_b(par):
        pltpu.make_async_copy(sh_hbm.at[pl.ds(0, KA)], vas[par], gsems[par]).wait()
        pltpu.make_async_copy(sh_hbm.at[pl.ds(0, KA)], vbs[par], gsems[par]).wait()

    issue_wave_a(0, 0)
    wait_wave_a(0)
    issue_wave_b(0)
    issue_wave_a(1, 1)

    def pair(p, carry):
        for par in (0, 1):
            i = p * 2 + par
            bid = i * NT + wid
            nbid = bid + NT

            @pl.when(nbid < NBA)
            def _():
                wait_wave_a(1 - par)
                issue_wave_b(1 - par)

            @pl.when(bid < NBA)
            def _():
                wait_wave_b(par)
                va = vas[par]
                vb = vbs[par]
                sebv = sebvs[par]
                eev = eevs[par]
                pack = packs[par]

                @plsc.parallel_loop(0, KA // 16, unroll=4)
                def chunk(c):
                    sl = pl.ds(c * 16, 16)
                    s = va[sl] + vb[sl] + sebv[sl]
                    s = jnp.where(s >= 0.0, s, 0.01 * s)
                    ee = jnp.exp(s - m16)
                    eev[sl] = ee
                    plsc.addupdate_scatter(accd, [pack[2, sl]], ee)
                pltpu.sync_copy(eev, ee_out.at[pl.ds(bid * KA, KA)])

            issue_wave_a(i + 2, par)
        return carry

    lax.fori_loop(0, (nbt + 1) // 2, pair, 0)
    pltpu.sync_copy(accd, den_out.at[wid])


def _score_call(edata, sh, sf, seb, m, zn):
    fn = pl.kernel(
        _score_kernel,
        out_type=[
            jax.ShapeDtypeStruct((E,), _f32),
            jax.ShapeDtypeStruct((NT, NPAD), _f32),
        ],
        mesh=_mesh(),
        compiler_params=_SC_PARAMS,
        scratch_types=[
            pltpu.VMEM((NPAD,), _f32),
            pltpu.VMEM((4, KA), _i32),
            pltpu.VMEM((4, KA), _i32),
            pltpu.VMEM((KA,), _f32),
            pltpu.VMEM((KA,), _f32),
            pltpu.VMEM((KA,), _f32),
            pltpu.VMEM((KA,), _f32),
            pltpu.VMEM((KA,), _f32),
            pltpu.VMEM((KA,), _f32),
            pltpu.VMEM((KA,), _f32),
            pltpu.VMEM((KA,), _f32),
            pltpu.VMEM((128,), _f32),
            pltpu.SemaphoreType.DMA,
            pltpu.SemaphoreType.DMA,
            pltpu.SemaphoreType.DMA,
            pltpu.SemaphoreType.DMA,
        ],
    )
    return fn(edata, sh, sf, seb, m, zn)


KA = 64
NBA = E // KA
NSC = 10112
ROWS_AGG = NSC // 16


def _agg_kernel(edata_hbm, ee_hbm, dinv_hbm, ph_hbm, pf_hbm,
                b_hbm, zb_hbm, out_hbm,
                agg_sh, pack0, pack1, av0, av1, dvv0, dvv1,
                phv0, phv1, pfv0, pfv1, bv,
                lsem0, lsem1, gsem0, gsem1, bsem):
    packs = (pack0, pack1)
    avs = (av0, av1)
    dvvs = (dvv0, dvv1)
    phvs = (phv0, phv1)
    pfvs = (pfv0, pfv1)
    lsems = (lsem0, lsem1)
    gsems = (gsem0, gsem1)
    cid = lax.axis_index("c")
    sid = lax.axis_index("s")
    wid = sid * 2 + cid
    pltpu.sync_copy(zb_hbm, agg_sh.at[pl.ds(sid * ROWS_AGG, ROWS_AGG)])
    plsc.subcore_barrier()

    nbt = (NBA + NT - 1) // NT

    def issue_wave_a(i, par):
        bid = i * NT + wid

        @pl.when(bid < NBA)
        def _():
            pltpu.async_copy(edata_hbm.at[bid], packs[par], lsems[par])
            pltpu.async_copy(ee_hbm.at[pl.ds(bid * KA, KA)], avs[par], lsems[par])

    def wait_wave_a(par):
        pltpu.make_async_copy(edata_hbm.at[0], packs[par], lsems[par]).wait()
        pltpu.make_async_copy(ee_hbm.at[pl.ds(0, KA)], avs[par], lsems[par]).wait()

    def issue_wave_b(i, par):
        pltpu.async_copy(ph_hbm.at[packs[par].at[0]], phvs[par], gsems[par])
        pltpu.async_copy(pf_hbm.at[packs[par].at[1]], pfvs[par], gsems[par])
        pltpu.async_copy(dinv_hbm.at[packs[par].at[2]], dvvs[par], gsems[par])

    def wait_wave_b(par):
        pltpu.make_async_copy(ph_hbm.at[pl.ds(0, KA)], phvs[par], gsems[par]).wait()
        pltpu.make_async_copy(ph_hbm.at[pl.ds(0, KA)], pfvs[par], gsems[par]).wait()
        pltpu.make_async_copy(dinv_hbm.at[pl.ds(0, KA)], dvvs[par], gsems[par]).wait()

    issue_wave_a(0, 0)
    wait_wave_a(0)
    issue_wave_b(0, 0)
    issue_wave_a(1, 1)

    def pair(p, carry):
        for par in (0, 1):
            i = p * 2 + par
            bid = i * NT + wid
            nbid = bid + NT

            @pl.when(nbid < NBA)
            def _():
                wait_wave_a(1 - par)
                issue_wave_b(i + 1, 1 - par)

            @pl.when(bid < NBA)
            def _():
                pltpu.async_copy(b_hbm.at[pl.ds(bid * KA, KA)], bv, bsem)
                wait_wave_b(par)
                av = avs[par]
                dvv = dvvs[par]
                phv = phvs[par]
                pfv = pfvs[par]

                @plsc.parallel_loop(0, KA // 16, unroll=4)
                def mka(c):
                    sl = pl.ds(c * 16, 16)
                    av[sl] = av[sl] * dvv[sl]
                pltpu.make_async_copy(b_hbm.at[pl.ds(0, KA)], bv, bsem).wait()

                @plsc.parallel_loop(0, KA, unroll=8)
                def row(r):
                    a16 = plsc.load_gather(av, [jnp.full((16,), 0, _i32) + r])
                    ph_r = phv.at[r]
                    pf_r = pfv.at[r]
                    b_r = bv.at[r]
                    for c in range(8):
                        sl = pl.ds(c * 16, 16)
                        ph_r[sl] = a16 * (ph_r[sl] + pf_r[sl] + b_r[sl])
                pltpu.sync_copy(phv, agg_sh.at[packs[par].at[2]], add=True)

            issue_wave_a(i + 2, par)
        return carry

    lax.fori_loop(0, (nbt + 1) // 2, pair, 0)
    plsc.subcore_barrier()
    sl = pl.ds(sid * ROWS_AGG, ROWS_AGG)
    pltpu.sync_copy(agg_sh.at[sl], out_hbm.at[cid, sl])


def _agg_call(edata, ee, dinv, ph, pf, btab, zb):
    fn = pl.kernel(
        _agg_kernel,
        out_type=jax.ShapeDtypeStruct((2, NSC, 128), _f32),
        mesh=_mesh(),
        compiler_params=_SC_PARAMS,
        scratch_types=[
            pltpu.VMEM_SHARED((NSC, 128), _f32),
            pltpu.VMEM((4, KA), _i32),
            pltpu.VMEM((4, KA), _i32),
            pltpu.VMEM((KA,), _f32),
            pltpu.VMEM((KA,), _f32),
            pltpu.VMEM((KA,), _f32),
            pltpu.VMEM((KA,), _f32),
            pltpu.VMEM((KA, 128), _f32),
            pltpu.VMEM((KA, 128), _f32),
            pltpu.VMEM((KA, 128), _f32),
            pltpu.VMEM((KA, 128), _f32),
            pltpu.VMEM((KA, 128), _f32),
            pltpu.SemaphoreType.DMA,
            pltpu.SemaphoreType.DMA,
            pltpu.SemaphoreType.DMA,
            pltpu.SemaphoreType.DMA,
            pltpu.SemaphoreType.DMA,
        ],
    )
    return fn(edata, ee, dinv, ph, pf, btab, zb)


def _combine_call(parts, mode):
    def body(p_ref, o_ref):
        s = jnp.sum(p_ref[...], axis=0, keepdims=True)
        if mode == "deg":
            o_ref[...] = lax.rsqrt(jnp.maximum(s, 1.0))
        else:
            o_ref[...] = 1.0 / (s + 1e-9)

    nb = NPAD // 1024
    return pl.pallas_call(
        body,
        grid=(nb,),
        in_specs=[pl.BlockSpec((NT, 1024), lambda j: (0, j))],
        out_specs=pl.BlockSpec((1, 1024), lambda j: (0, j)),
        out_shape=jax.ShapeDtypeStruct((1, NPAD), _f32),
    )(parts)


_BN = 400


def _tables_call(x, d_col, wlo, whi, amlo, amhi, af, hout):
    din = x.shape[1]
    has_hi = hout == 256

    def body(*refs):
        if has_hi:
            (x_ref, d_ref, wlo_ref, whi_ref, amlo_ref, amhi_ref, af_ref,
             plo_ref, phi_ref, s_ref) = refs
        else:
            x_ref, d_ref, wlo_ref, amlo_ref, af_ref, plo_ref, s_ref = refs
        xs = x_ref[...] * d_ref[...]
        lo = jnp.dot(xs, wlo_ref[...], preferred_element_type=_f32)
        plo_ref[...] = lo
        s = jnp.dot(lo, amlo_ref[...], preferred_element_type=_f32)
        s = s + jnp.dot(xs, af_ref[...], preferred_element_type=_f32)
        if has_hi:
            hi = jnp.dot(xs, whi_ref[...], preferred_element_type=_f32)
            phi_ref[...] = hi
            s = s + jnp.dot(hi, amhi_ref[...], preferred_element_type=_f32)
        s_ref[...] = s

    wspec = pl.BlockSpec((din, R * 128), lambda j: (0, 0))
    amspec = pl.BlockSpec((R * 128, R), lambda j: (0, 0))
    in_specs = [
        pl.BlockSpec((_BN, din), lambda j: (j, 0)),
        pl.BlockSpec((_BN, 1), lambda j: (j, 0)),
        wspec,
    ]
    out_shapes = [jax.ShapeDtypeStruct((N, R * 128), _f32)]
    out_specs = [pl.BlockSpec((_BN, R * 128), lambda j: (j, 0))]
    args = [x, d_col, wlo]
    if has_hi:
        in_specs.append(wspec)
        args.append(whi)
        out_shapes.append(jax.ShapeDtypeStruct((N, R * 128), _f32))
        out_specs.append(pl.BlockSpec((_BN, R * 128), lambda j: (j, 0)))
    in_specs.append(amspec)
    args.append(amlo)
    if has_hi:
        in_specs.append(amspec)
        args.append(amhi)
    in_specs.append(pl.BlockSpec((din, R), lambda j: (0, 0)))
    args.append(af)
    out_shapes.append(jax.ShapeDtypeStruct((N, R), _f32))
    out_specs.append(pl.BlockSpec((_BN, R), lambda j: (j, 0)))

    res = pl.pallas_call(
        body,
        grid=(N // _BN,),
        in_specs=in_specs,
        out_specs=out_specs,
        out_shape=out_shapes,
    )(*args)
    if has_hi:
        return res[0], res[1], res[2]
    return res[0], None, res[1]


_BE = 640


def _btab_call(efeat, eidc, we_bf, amlo_c, amhi_c, hout):
    has_hi = hout == 256

    def body(*refs):
        if has_hi:
            (ef_ref, eidc_ref, w_ref, amlo_ref, amhi_ref,
             blo_ref, bhi_ref, seb_ref) = refs
        else:
            ef_ref, eidc_ref, w_ref, amlo_ref, blo_ref, seb_ref = refs
        ef = ef_ref[...]
        eidc = eidc_ref[...]
        acc = jnp.zeros((_BE, hout), _f32)
        for r in range(R):
            msk = (eidc == float(r)).astype(_f32)
            efm = (ef * msk).astype(jnp.bfloat16)
            acc = acc + jnp.dot(efm, w_ref[r], preferred_element_type=_f32)
        blo = acc[:, :128]
        blo_ref[...] = blo
        s = jnp.dot(blo, amlo_ref[...], preferred_element_type=_f32)
        if has_hi:
            bhi = acc[:, 128:]
            bhi_ref[...] = bhi
            s = s + jnp.dot(bhi, amhi_ref[...], preferred_element_type=_f32)
        seb_ref[...] = s

    in_specs = [
        pl.BlockSpec((_BE, D), lambda j: (j, 0)),
        pl.BlockSpec((_BE, 1), lambda j: (j, 0)),
        pl.BlockSpec((R, D, hout), lambda j: (0, 0, 0)),
        pl.BlockSpec((128, 1), lambda j: (0, 0)),
    ]
    args = [efeat, eidc, we_bf, amlo_c]
    if has_hi:
        in_specs.append(pl.BlockSpec((128, 1), lambda j: (0, 0)))
        args.append(amhi_c)
    out_shapes = [jax.ShapeDtypeStruct((E, 128), _f32)]
    out_specs = [pl.BlockSpec((_BE, 128), lambda j: (j, 0))]
    if has_hi:
        out_shapes.append(jax.ShapeDtypeStruct((E, 128), _f32))
        out_specs.append(pl.BlockSpec((_BE, 128), lambda j: (j, 0)))
    out_shapes.append(jax.ShapeDtypeStruct((E, 1), _f32))
    out_specs.append(pl.BlockSpec((_BE, 1), lambda j: (j, 0)))

    res = pl.pallas_call(
        body,
        grid=(E // _BE,),
        in_specs=in_specs,
        out_specs=out_specs,
        out_shape=out_shapes,
    )(*args)
    if has_hi:
        return res[0], res[1], res[2]
    return res[0], None, res[1]


def _stats_call(s_h, s_f, seb_col):
    def body(sh_ref, sf_ref, seb_ref, m_ref):
        m = jnp.max(sh_ref[...]) + jnp.max(sf_ref[...]) + jnp.max(seb_ref[...])
        m_ref[...] = jnp.full((1, 128), jnp.maximum(m, 0.0), _f32)

    return pl.pallas_call(
        body,
        out_shape=jax.ShapeDtypeStruct((1, 128), _f32),
    )(s_h, s_f, seb_col)


def _epilogue_call(x, loopw, dsi_col, bias, agg_lo, agg_hi, hout):
    has_hi = agg_hi is not None

    def body(*refs):
        if has_hi:
            x_ref, lw_ref, d_ref, b_ref, alo_ref, ahi_ref, o_ref = refs
        else:
            x_ref, lw_ref, d_ref, b_ref, alo_ref, o_ref = refs
        agg = alo_ref[0] + alo_ref[1]
        if has_hi:
            agg = jnp.concatenate([agg, ahi_ref[0] + ahi_ref[1]], axis=1)
        hh = agg + jnp.dot(x_ref[...], lw_ref[...], preferred_element_type=_f32)
        o = hh * d_ref[...] + b_ref[...]
        o_ref[...] = jnp.maximum(o, 0.0)

    in_specs = [
        pl.BlockSpec((_BN, D), lambda j: (j, 0)),
        pl.BlockSpec((D, hout), lambda j: (0, 0)),
        pl.BlockSpec((_BN, 1), lambda j: (j, 0)),
        pl.BlockSpec((1, hout), lambda j: (0, 0)),
        pl.BlockSpec((2, _BN, 128), lambda j: (0, j, 0)),
    ]
    args = [x, loopw, dsi_col, bias, agg_lo]
    if has_hi:
        in_specs.append(pl.BlockSpec((2, _BN, 128), lambda j: (0, j, 0)))
        args.append(agg_hi)

    return pl.pallas_call(
        body,
        grid=(N // _BN,),
        in_specs=in_specs,
        out_specs=pl.BlockSpec((_BN, hout), lambda j: (j, 0)),
        out_shape=jax.ShapeDtypeStruct((N, hout), _f32),
    )(*args)


def _am_mat(attn_m_half):
    return (jnp.eye(R, dtype=_f32)[:, None, :] * attn_m_half[None, :, None]
            ).reshape(R * 128, R)


def _layer(h, x, edata, eidc, efeat, W, attn, loopw, bias,
           dso_c, dsi_c, ones_c, zn, zb):
    dh = h.shape[1]
    hout = W.shape[2]
    Wh, We, Wf = W[:, :dh, :], W[:, dh:dh + D, :], W[:, dh + D:, :]
    attn_f, attn_m = attn[:D], attn[D:]
    has_hi = hout == 256

    wh_lo = Wh[:, :, :128].transpose(1, 0, 2).reshape(dh, R * 128)
    wf_lo = Wf[:, :, :128].transpose(1, 0, 2).reshape(D, R * 128)
    wh_hi = Wh[:, :, 128:].transpose(1, 0, 2).reshape(dh, R * 128) if has_hi else None
    wf_hi = Wf[:, :, 128:].transpose(1, 0, 2).reshape(D, R * 128) if has_hi else None
    amlo = _am_mat(attn_m[:128])
    amhi = _am_mat(attn_m[128:]) if has_hi else None
    af = attn_f[:, None] * jnp.ones((1, R), _f32)
    zf = jnp.zeros((dh, R), _f32)
    amlo_c = attn_m[:128][:, None]
    amhi_c = attn_m[128:][:, None] if has_hi else None

    plo, phi, s_h = _tables_call(h, dso_c, wh_lo, wh_hi, amlo, amhi, zf, hout)
    flo, fhi, s_f = _tables_call(x, ones_c, wf_lo, wf_hi, amlo, amhi, af, hout)
    blo, bhi, seb_col = _btab_call(efeat, eidc, We.astype(jnp.bfloat16),
                                   amlo_c, amhi_c, hout)

    m = _stats_call(s_h, s_f, seb_col.reshape(1, E)).reshape(128)
    ee, den = _score_call(edata, s_h.reshape(N * R), s_f.reshape(N * R),
                          seb_col.reshape(E), m, zn)
    dinv = _combine_call(den, "den").reshape(NPAD)

    agg_lo = _agg_call(edata, ee, dinv,
                       plo.reshape(N * R, 128), flo.reshape(N * R, 128), blo, zb)
    agg_hi = None
    if has_hi:
        agg_hi = _agg_call(edata, ee, dinv,
                           phi.reshape(N * R, 128), fhi.reshape(N * R, 128), bhi, zb)
    return _epilogue_call(x, loopw, dsi_c, bias[None, :], agg_lo, agg_hi, hout)


def kernel(x, edge_index, edge_feat, edge_type, W1, attn1, loop1, bias1,
           W2, attn2, loop2, bias2):
    src = edge_index[0].astype(_i32)
    dst = edge_index[1].astype(_i32)
    eid = edge_type.astype(_i32)
    eidc = edge_type.astype(_f32)[:, None]
    zn = jnp.zeros((NPAD,), _f32)
    zb = jnp.zeros((NSC // 16, 128), _f32)
    ones_c = jnp.ones((N, 1), _f32)

    po, pi, edata = _deg_call(src, dst, eid, zn)
    dso_c = _combine_call(po, "deg").reshape(NPAD, 1)[:N]
    dsi_c = _combine_call(pi, "deg").reshape(NPAD, 1)[:N]

    h1 = _layer(x, x, edata, eidc, edge_feat, W1, attn1, loop1, bias1,
                dso_c, dsi_c, ones_c, zn, zb)
    h2 = _layer(h1, x, edata, eidc, edge_feat, W2, attn2, loop2, bias2,
                dso_c, dsi_c, ones_c, zn, zb)
    return h2


# speedup vs baseline: 1.0523x; 1.0523x over previous
import jax
import jax.numpy as jnp
from jax import lax
from jax.experimental import pallas as pl
from jax.experimental.pallas import tpu as pltpu
from jax.experimental.pallas import tpu_sc as plsc

N = 10000
E = 160000
D = 128
R = 8
NPAD = 10240
NT = 32
K = 128
NB = E // K

_f32 = jnp.float32
_i32 = jnp.int32


def _mesh():
    return plsc.VectorSubcoreMesh(core_axis_name="c", subcore_axis_name="s")


_SC_PARAMS = pltpu.CompilerParams(needs_layout_passes=False)


def _deg_kernel(src_hbm, dst_hbm, eid_hbm, zn_hbm, out_o, out_i, edata_out,
                acc_o, acc_i, srcv, dstv, eidv, packa, packb, sem):
    wid = lax.axis_index("s") * 2 + lax.axis_index("c")
    pltpu.sync_copy(zn_hbm, acc_o)
    pltpu.sync_copy(zn_hbm, acc_i)
    ones = jnp.full((16,), 1.0, _f32)

    def batch(b, carry):
        bid = b * NT + wid

        @pl.when(bid < NB)
        def _():
            base = bid * K
            pltpu.sync_copy(src_hbm.at[pl.ds(base, K)], srcv)
            pltpu.sync_copy(dst_hbm.at[pl.ds(base, K)], dstv)
            pltpu.sync_copy(eid_hbm.at[pl.ds(base, K)], eidv)

            for c in range(K // 16):
                sl = pl.ds(c * 16, 16)
                hsl = pl.ds((c % 4) * 16, 16)
                s16 = srcv[sl]
                d16 = dstv[sl]
                e16 = eidv[sl]
                plsc.addupdate_scatter(acc_o, [s16], ones)
                plsc.addupdate_scatter(acc_i, [d16], ones)
                pk = packa if c < 4 else packb
                pk[0, hsl] = s16 * R + e16
                pk[1, hsl] = d16 * R + e16
                pk[2, hsl] = d16
                pk[3, hsl] = s16

            pltpu.sync_copy(packa, edata_out.at[2 * bid])
            pltpu.sync_copy(packb, edata_out.at[2 * bid + 1])

        return carry

    lax.fori_loop(0, (NB + NT - 1) // NT, batch, 0)
    pltpu.sync_copy(acc_o, out_o.at[wid])
    pltpu.sync_copy(acc_i, out_i.at[wid])


def _deg_call(src, dst, eid, zn):
    fn = pl.kernel(
        _deg_kernel,
        out_type=[
            jax.ShapeDtypeStruct((NT, NPAD), _f32),
            jax.ShapeDtypeStruct((NT, NPAD), _f32),
            jax.ShapeDtypeStruct((NBA, 4, KA), _i32),
        ],
        mesh=_mesh(),
        compiler_params=_SC_PARAMS,
        scratch_types=[
            pltpu.VMEM((NPAD,), _f32),
            pltpu.VMEM((NPAD,), _f32),
            pltpu.VMEM((K,), _i32),
            pltpu.VMEM((K,), _i32),
            pltpu.VMEM((K,), _i32),
            pltpu.VMEM((4, KA), _i32),
            pltpu.VMEM((4, KA), _i32),
            pltpu.SemaphoreType.DMA,
        ],
    )
    return fn(src, dst, eid, zn)


def _score_kernel(edata_hbm, sh_hbm, sf_hbm, seb_hbm, m_hbm,
                  zn_hbm, ee_out, den_out,
                  accd, pack0, pack1, va0, va1, vb0, vb1, sebv0, sebv1,
                  eev0, eev1, mv, lsem0, lsem1, gsem0, gsem1):
    packs = (pack0, pack1)
    vas = (va0, va1)
    vbs = (vb0, vb1)
    sebvs = (sebv0, sebv1)
    eevs = (eev0, eev1)
    lsems = (lsem0, lsem1)
    gsems = (gsem0, gsem1)
    wid = lax.axis_index("s") * 2 + lax.axis_index("c")
    pltpu.sync_copy(zn_hbm, accd)
    pltpu.sync_copy(m_hbm, mv)
    m16 = mv[pl.ds(0, 16)]

    nbt = (NBA + NT - 1) // NT

    def issue_wave_a(i, par):
        bid = i * NT + wid

        @pl.when(bid < NBA)
        def _():
            pltpu.async_copy(edata_hbm.at[bid], packs[par], lsems[par])
            pltpu.async_copy(seb_hbm.at[pl.ds(bid * KA, KA)], sebvs[par],
                             lsems[par])

    def wait_wave_a(par):
        pltpu.make_async_copy(edata_hbm.at[0], packs[par], lsems[par]).wait()
        pltpu.make_async_copy(seb_hbm.at[pl.ds(0, KA)], sebvs[par],
                              lsems[par]).wait()

    def issue_wave_b(par):
        pltpu.async_copy(sh_hbm.at[packs[par].at[0]], vas[par], gsems[par])
        pltpu.async_copy(sf_hbm.at[packs[par].at[1]], vbs[par], gsems[par])

    def wait_wave_b(par):
        pltpu.make_async_copy(sh_hbm.at[pl.ds(0, KA)], vas[par], gsems[par]).wait()
        pltpu.make_async_copy(sh_hbm.at[pl.ds(0, KA)], vbs[par], gsems[par]).wait()

    issue_wave_a(0, 0)
    wait_wave_a(0)
    issue_wave_b(0)
    issue_wave_a(1, 1)

    def pair(p, carry):
        for par in (0, 1):
            i = p * 2 + par
            bid = i * NT + wid
            nbid = bid + NT

            @pl.when(nbid < NBA)
            def _():
                wait_wave_a(1 - par)
                issue_wave_b(1 - par)

            @pl.when(bid < NBA)
            def _():
                wait_wave_b(par)
                va = vas[par]
                vb = vbs[par]
                sebv = sebvs[par]
                eev = eevs[par]
                pack = packs[par]

                @plsc.parallel_loop(0, KA // 16, unroll=4)
                def chunk(c):
                    sl = pl.ds(c * 16, 16)
                    s = va[sl] + vb[sl] + sebv[sl]
                    s = jnp.where(s >= 0.0, s, 0.01 * s)
                    ee = jnp.exp(s - m16)
                    eev[sl] = ee
                    plsc.addupdate_scatter(accd, [pack[2, sl]], ee)
                pltpu.sync_copy(eev, ee_out.at[pl.ds(bid * KA, KA)])

            issue_wave_a(i + 2, par)
        return carry

    lax.fori_loop(0, (nbt + 1) // 2, pair, 0)
    pltpu.sync_copy(accd, den_out.at[wid])


def _score_call(edata, sh, sf, seb, m, zn):
    fn = pl.kernel(
        _score_kernel,
        out_type=[
            jax.ShapeDtypeStruct((E,), _f32),
            jax.ShapeDtypeStruct((NT, NPAD), _f32),
        ],
        mesh=_mesh(),
        compiler_params=_SC_PARAMS,
        scratch_types=[
            pltpu.VMEM((NPAD,), _f32),
            pltpu.VMEM((4, KA), _i32),
            pltpu.VMEM((4, KA), _i32),
            pltpu.VMEM((KA,), _f32),
            pltpu.VMEM((KA,), _f32),
            pltpu.VMEM((KA,), _f32),
            pltpu.VMEM((KA,), _f32),
            pltpu.VMEM((KA,), _f32),
            pltpu.VMEM((KA,), _f32),
            pltpu.VMEM((KA,), _f32),
            pltpu.VMEM((KA,), _f32),
            pltpu.VMEM((128,), _f32),
            pltpu.SemaphoreType.DMA,
            pltpu.SemaphoreType.DMA,
            pltpu.SemaphoreType.DMA,
            pltpu.SemaphoreType.DMA,
        ],
    )
    return fn(edata, sh, sf, seb, m, zn)


KA = 64
NBA = E // KA
NSC = 10112
ROWS_AGG = NSC // 16


def _agg_kernel(edata_hbm, ee_hbm, dinv_hbm, ph_hbm, pf_hbm,
                b_hbm, zb_hbm, out_hbm,
                agg_sh, pack0, pack1, av0, av1, dvv0, dvv1,
                phv0, phv1, pfv0, pfv1, bv,
                lsem0, lsem1, gsem0, gsem1, bsem):
    packs = (pack0, pack1)
    avs = (av0, av1)
    dvvs = (dvv0, dvv1)
    phvs = (phv0, phv1)
    pfvs = (pfv0, pfv1)
    lsems = (lsem0, lsem1)
    gsems = (gsem0, gsem1)
    cid = lax.axis_index("c")
    sid = lax.axis_index("s")
    wid = sid * 2 + cid
    pltpu.sync_copy(zb_hbm, agg_sh.at[pl.ds(sid * ROWS_AGG, ROWS_AGG)])
    plsc.subcore_barrier()

    nbt = (NBA + NT - 1) // NT

    def issue_wave_a(i, par):
        bid = i * NT + wid

        @pl.when(bid < NBA)
        def _():
            pltpu.async_copy(edata_hbm.at[bid], packs[par], lsems[par])
            pltpu.async_copy(ee_hbm.at[pl.ds(bid * KA, KA)], avs[par], lsems[par])

    def wait_wave_a(par):
        pltpu.make_async_copy(edata_hbm.at[0], packs[par], lsems[par]).wait()
        pltpu.make_async_copy(ee_hbm.at[pl.ds(0, KA)], avs[par], lsems[par]).wait()

    def issue_wave_b(i, par):
        pltpu.async_copy(ph_hbm.at[packs[par].at[0]], phvs[par], gsems[par])
        pltpu.async_copy(pf_hbm.at[packs[par].at[1]], pfvs[par], gsems[par])
        pltpu.async_copy(dinv_hbm.at[packs[par].at[2]], dvvs[par], gsems[par])

    def wait_wave_b(par):
        pltpu.make_async_copy(ph_hbm.at[pl.ds(0, KA)], phvs[par], gsems[par]).wait()
        pltpu.make_async_copy(ph_hbm.at[pl.ds(0, KA)], pfvs[par], gsems[par]).wait()
        pltpu.make_async_copy(dinv_hbm.at[pl.ds(0, KA)], dvvs[par], gsems[par]).wait()

    issue_wave_a(0, 0)
    wait_wave_a(0)
    issue_wave_b(0, 0)
    issue_wave_a(1, 1)

    def pair(p, carry):
        for par in (0, 1):
            i = p * 2 + par
            bid = i * NT + wid
            nbid = bid + NT

            @pl.when(nbid < NBA)
            def _():
                wait_wave_a(1 - par)
                issue_wave_b(i + 1, 1 - par)

            @pl.when(bid < NBA)
            def _():
                pltpu.async_copy(b_hbm.at[pl.ds(bid * KA, KA)], bv, bsem)
                wait_wave_b(par)
                av = avs[par]
                dvv = dvvs[par]
                phv = phvs[par]
                pfv = pfvs[par]

                @plsc.parallel_loop(0, KA // 16, unroll=4)
                def mka(c):
                    sl = pl.ds(c * 16, 16)
                    av[sl] = av[sl] * dvv[sl]
                pltpu.make_async_copy(b_hbm.at[pl.ds(0, KA)], bv, bsem).wait()

                @plsc.parallel_loop(0, KA, unroll=4)
                def row(r):
                    a16 = plsc.load_gather(av, [jnp.full((16,), 0, _i32) + r])
                    ph_r = phv.at[r]
                    pf_r = pfv.at[r]
                    b_r = bv.at[r]
                    for c in range(8):
                        sl = pl.ds(c * 16, 16)
                        ph_r[sl] = a16 * (ph_r[sl] + pf_r[sl] + b_r[sl])
                pltpu.sync_copy(phv, agg_sh.at[packs[par].at[2]], add=True)

            issue_wave_a(i + 2, par)
        return carry

    lax.fori_loop(0, (nbt + 1) // 2, pair, 0)
    plsc.subcore_barrier()
    sl = pl.ds(sid * ROWS_AGG, ROWS_AGG)
    pltpu.sync_copy(agg_sh.at[sl], out_hbm.at[cid, sl])


def _agg_call(edata, ee, dinv, ph, pf, btab, zb):
    fn = pl.kernel(
        _agg_kernel,
        out_type=jax.ShapeDtypeStruct((2, NSC, 128), _f32),
        mesh=_mesh(),
        compiler_params=_SC_PARAMS,
        scratch_types=[
            pltpu.VMEM_SHARED((NSC, 128), _f32),
            pltpu.VMEM((4, KA), _i32),
            pltpu.VMEM((4, KA), _i32),
            pltpu.VMEM((KA,), _f32),
            pltpu.VMEM((KA,), _f32),
            pltpu.VMEM((KA,), _f32),
            pltpu.VMEM((KA,), _f32),
            pltpu.VMEM((KA, 128), _f32),
            pltpu.VMEM((KA, 128), _f32),
            pltpu.VMEM((KA, 128), _f32),
            pltpu.VMEM((KA, 128), _f32),
            pltpu.VMEM((KA, 128), _f32),
            pltpu.SemaphoreType.DMA,
            pltpu.SemaphoreType.DMA,
            pltpu.SemaphoreType.DMA,
            pltpu.SemaphoreType.DMA,
            pltpu.SemaphoreType.DMA,
        ],
    )
    return fn(edata, ee, dinv, ph, pf, btab, zb)


def _combine_call(parts, mode):
    def body(p_ref, o_ref):
        s = jnp.sum(p_ref[...], axis=0, keepdims=True)
        if mode == "deg":
            o_ref[...] = lax.rsqrt(jnp.maximum(s, 1.0))
        else:
            o_ref[...] = 1.0 / (s + 1e-9)

    nb = NPAD // 1024
    return pl.pallas_call(
        body,
        grid=(nb,),
        in_specs=[pl.BlockSpec((NT, 1024), lambda j: (0, j))],
        out_specs=pl.BlockSpec((1, 1024), lambda j: (0, j)),
        out_shape=jax.ShapeDtypeStruct((1, NPAD), _f32),
    )(parts)


_BN = 400


def _tables_call(x, d_col, wlo, whi, amlo, amhi, af, hout):
    din = x.shape[1]
    has_hi = hout == 256

    def body(*refs):
        if has_hi:
            (x_ref, d_ref, wlo_ref, whi_ref, amlo_ref, amhi_ref, af_ref,
             plo_ref, phi_ref, s_ref) = refs
        else:
            x_ref, d_ref, wlo_ref, amlo_ref, af_ref, plo_ref, s_ref = refs
        xs = x_ref[...] * d_ref[...]
        lo = jnp.dot(xs, wlo_ref[...], preferred_element_type=_f32)
        plo_ref[...] = lo
        s = jnp.dot(lo, amlo_ref[...], preferred_element_type=_f32)
        s = s + jnp.dot(xs, af_ref[...], preferred_element_type=_f32)
        if has_hi:
            hi = jnp.dot(xs, whi_ref[...], preferred_element_type=_f32)
            phi_ref[...] = hi
            s = s + jnp.dot(hi, amhi_ref[...], preferred_element_type=_f32)
        s_ref[...] = s

    wspec = pl.BlockSpec((din, R * 128), lambda j: (0, 0))
    amspec = pl.BlockSpec((R * 128, R), lambda j: (0, 0))
    in_specs = [
        pl.BlockSpec((_BN, din), lambda j: (j, 0)),
        pl.BlockSpec((_BN, 1), lambda j: (j, 0)),
        wspec,
    ]
    out_shapes = [jax.ShapeDtypeStruct((N, R * 128), _f32)]
    out_specs = [pl.BlockSpec((_BN, R * 128), lambda j: (j, 0))]
    args = [x, d_col, wlo]
    if has_hi:
        in_specs.append(wspec)
        args.append(whi)
        out_shapes.append(jax.ShapeDtypeStruct((N, R * 128), _f32))
        out_specs.append(pl.BlockSpec((_BN, R * 128), lambda j: (j, 0)))
    in_specs.append(amspec)
    args.append(amlo)
    if has_hi:
        in_specs.append(amspec)
        args.append(amhi)
    in_specs.append(pl.BlockSpec((din, R), lambda j: (0, 0)))
    args.append(af)
    out_shapes.append(jax.ShapeDtypeStruct((N, R), _f32))
    out_specs.append(pl.BlockSpec((_BN, R), lambda j: (j, 0)))

    res = pl.pallas_call(
        body,
        grid=(N // _BN,),
        in_specs=in_specs,
        out_specs=out_specs,
        out_shape=out_shapes,
    )(*args)
    if has_hi:
        return res[0], res[1], res[2]
    return res[0], None, res[1]


_BE = 640


def _btab_call(efeat, eidc, we_bf, amlo_c, amhi_c, hout):
    has_hi = hout == 256

    def body(*refs):
        if has_hi:
            (ef_ref, eidc_ref, w_ref, amlo_ref, amhi_ref,
             blo_ref, bhi_ref, seb_ref) = refs
        else:
            ef_ref, eidc_ref, w_ref, amlo_ref, blo_ref, seb_ref = refs
        ef = ef_ref[...]
        eidc = eidc_ref[...]
        acc = jnp.zeros((_BE, hout), _f32)
        for r in range(R):
            msk = (eidc == float(r)).astype(_f32)
            efm = (ef * msk).astype(jnp.bfloat16)
            acc = acc + jnp.dot(efm, w_ref[r], preferred_element_type=_f32)
        blo = acc[:, :128]
        blo_ref[...] = blo
        s = jnp.dot(blo, amlo_ref[...], preferred_element_type=_f32)
        if has_hi:
            bhi = acc[:, 128:]
            bhi_ref[...] = bhi
            s = s + jnp.dot(bhi, amhi_ref[...], preferred_element_type=_f32)
        seb_ref[...] = s

    in_specs = [
        pl.BlockSpec((_BE, D), lambda j: (j, 0)),
        pl.BlockSpec((_BE, 1), lambda j: (j, 0)),
        pl.BlockSpec((R, D, hout), lambda j: (0, 0, 0)),
        pl.BlockSpec((128, 1), lambda j: (0, 0)),
    ]
    args = [efeat, eidc, we_bf, amlo_c]
    if has_hi:
        in_specs.append(pl.BlockSpec((128, 1), lambda j: (0, 0)))
        args.append(amhi_c)
    out_shapes = [jax.ShapeDtypeStruct((E, 128), _f32)]
    out_specs = [pl.BlockSpec((_BE, 128), lambda j: (j, 0))]
    if has_hi:
        out_shapes.append(jax.ShapeDtypeStruct((E, 128), _f32))
        out_specs.append(pl.BlockSpec((_BE, 128), lambda j: (j, 0)))
    out_shapes.append(jax.ShapeDtypeStruct((E, 1), _f32))
    out_specs.append(pl.BlockSpec((_BE, 1), lambda j: (j, 0)))

    res = pl.pallas_call(
        body,
        grid=(E // _BE,),
        in_specs=in_specs,
        out_specs=out_specs,
        out_shape=out_shapes,
    )(*args)
    if has_hi:
        return res[0], res[1], res[2]
    return res[0], None, res[1]


def _stats_call(s_h, s_f, seb_col):
    def body(sh_ref, sf_ref, seb_ref, m_ref):
        m = jnp.max(sh_ref[...]) + jnp.max(sf_ref[...]) + jnp.max(seb_ref[...])
        m_ref[...] = jnp.full((1, 128), jnp.maximum(m, 0.0), _f32)

    return pl.pallas_call(
        body,
        out_shape=jax.ShapeDtypeStruct((1, 128), _f32),
    )(s_h, s_f, seb_col)


def _epilogue_call(x, loopw, dsi_col, bias, agg_lo, agg_hi, hout):
    has_hi = agg_hi is not None

    def body(*refs):
        if has_hi:
            x_ref, lw_ref, d_ref, b_ref, alo_ref, ahi_ref, o_ref = refs
        else:
            x_ref, lw_ref, d_ref, b_ref, alo_ref, o_ref = refs
        agg = alo_ref[0] + alo_ref[1]
        if has_hi:
            agg = jnp.concatenate([agg, ahi_ref[0] + ahi_ref[1]], axis=1)
        hh = agg + jnp.dot(x_ref[...], lw_ref[...], preferred_element_type=_f32)
        o = hh * d_ref[...] + b_ref[...]
        o_ref[...] = jnp.maximum(o, 0.0)

    in_specs = [
        pl.BlockSpec((_BN, D), lambda j: (j, 0)),
        pl.BlockSpec((D, hout), lambda j: (0, 0)),
        pl.BlockSpec((_BN, 1), lambda j: (j, 0)),
        pl.BlockSpec((1, hout), lambda j: (0, 0)),
        pl.BlockSpec((2, _BN, 128), lambda j: (0, j, 0)),
    ]
    args = [x, loopw, dsi_col, bias, agg_lo]
    if has_hi:
        in_specs.append(pl.BlockSpec((2, _BN, 128), lambda j: (0, j, 0)))
        args.append(agg_hi)

    return pl.pallas_call(
        body,
        grid=(N // _BN,),
        in_specs=in_specs,
        out_specs=pl.BlockSpec((_BN, hout), lambda j: (j, 0)),
        out_shape=jax.ShapeDtypeStruct((N, hout), _f32),
    )(*args)


def _am_mat(attn_m_half):
    return (jnp.eye(R, dtype=_f32)[:, None, :] * attn_m_half[None, :, None]
            ).reshape(R * 128, R)


def _layer(h, x, edata, eidc, efeat, W, attn, loopw, bias,
           dso_c, dsi_c, ones_c, zn, zb):
    dh = h.shape[1]
    hout = W.shape[2]
    Wh, We, Wf = W[:, :dh, :], W[:, dh:dh + D, :], W[:, dh + D:, :]
    attn_f, attn_m = attn[:D], attn[D:]
    has_hi = hout == 256

    wh_lo = Wh[:, :, :128].transpose(1, 0, 2).reshape(dh, R * 128)
    wf_lo = Wf[:, :, :128].transpose(1, 0, 2).reshape(D, R * 128)
    wh_hi = Wh[:, :, 128:].transpose(1, 0, 2).reshape(dh, R * 128) if has_hi else None
    wf_hi = Wf[:, :, 128:].transpose(1, 0, 2).reshape(D, R * 128) if has_hi else None
    amlo = _am_mat(attn_m[:128])
    amhi = _am_mat(attn_m[128:]) if has_hi else None
    af = attn_f[:, None] * jnp.ones((1, R), _f32)
    zf = jnp.zeros((dh, R), _f32)
    amlo_c = attn_m[:128][:, None]
    amhi_c = attn_m[128:][:, None] if has_hi else None

    plo, phi, s_h = _tables_call(h, dso_c, wh_lo, wh_hi, amlo, amhi, zf, hout)
    flo, fhi, s_f = _tables_call(x, ones_c, wf_lo, wf_hi, amlo, amhi, af, hout)
    blo, bhi, seb_col = _btab_call(efeat, eidc, We.astype(jnp.bfloat16),
                                   amlo_c, amhi_c, hout)

    m = _stats_call(s_h, s_f, seb_col.reshape(1, E)).reshape(128)
    ee, den = _score_call(edata, s_h.reshape(N * R), s_f.reshape(N * R),
                          seb_col.reshape(E), m, zn)
    dinv = _combine_call(den, "den").reshape(NPAD)

    agg_lo = _agg_call(edata, ee, dinv,
                       plo.reshape(N * R, 128), flo.reshape(N * R, 128), blo, zb)
    agg_hi = None
    if has_hi:
        agg_hi = _agg_call(edata, ee, dinv,
                           phi.reshape(N * R, 128), fhi.reshape(N * R, 128), bhi, zb)
    return _epilogue_call(x, loopw, dsi_c, bias[None, :], agg_lo, agg_hi, hout)


def kernel(x, edge_index, edge_feat, edge_type, W1, attn1, loop1, bias1,
           W2, attn2, loop2, bias2):
    src = edge_index[0].astype(_i32)
    dst = edge_index[1].astype(_i32)
    eid = edge_type.astype(_i32)
    eidc = edge_type.astype(_f32)[:, None]
    zn = jnp.zeros((NPAD,), _f32)
    zb = jnp.zeros((NSC // 16, 128), _f32)
    ones_c = jnp.ones((N, 1), _f32)

    po, pi, edata = _deg_call(src, dst, eid, zn)
    dso_c = _combine_call(po, "deg").reshape(NPAD, 1)[:N]
    dsi_c = _combine_call(pi, "deg").reshape(NPAD, 1)[:N]

    h1 = _layer(x, x, edata, eidc, edge_feat, W1, attn1, loop1, bias1,
                dso_c, dsi_c, ones_c, zn, zb)
    h2 = _layer(h1, x, edata, eidc, edge_feat, W2, attn2, loop2, bias2,
                dso_c, dsi_c, ones_c, zn, zb)
    return h2
